# Initial kernel scaffold; baseline (speedup 1.0000x reference)
import jax
import jax.numpy as jnp
from jax.experimental import pallas as pl


def kernel(h, e, edge_index, W_fc, W_attn, W_eatt, W_ez):
    raise NotImplementedError("write your pallas kernel here")


# speedup vs baseline: 3.5174x; 3.5174x over previous
import functools

import jax
import jax.numpy as jnp
from jax import lax
from jax.experimental import pallas as pl
from jax.experimental.pallas import tpu as pltpu
from jax.experimental.pallas import tpu_sc as plsc

_NC = 2
_NS = 16
_L = 16
_NW = _NC * _NS

_C = 80
_QACC = 2560


def _tc_pre_body(h_ref, wfc_ref, wattn_ref, z_ref, s1_ref, s2_ref, m_ref):
    z = lax.dot_general(h_ref[...], wfc_ref[...], (((1,), (1,)), ((), ())),
                        preferred_element_type=jnp.float32)
    z_ref[...] = z
    wa = wattn_ref[...]
    w1 = wa[:, 0:128]
    w2 = wa[:, 128:256]
    s1 = lax.dot_general(z, w1, (((1,), (1,)), ((), ())),
                         preferred_element_type=jnp.float32)
    s2 = lax.dot_general(z, w2, (((1,), (1,)), ((), ())),
                         preferred_element_type=jnp.float32)
    s1_ref[...] = s1
    s2_ref[...] = s2
    m_ref[0, 0] = jnp.max(s1) + jnp.max(s2)


def _tc_pre(h, w_fc, w_attn):
    n = h.shape[0]
    return pl.pallas_call(
        _tc_pre_body,
        out_shape=(
            jax.ShapeDtypeStruct((n, 128), jnp.float32),
            jax.ShapeDtypeStruct((n, 1), jnp.float32),
            jax.ShapeDtypeStruct((n, 1), jnp.float32),
            jax.ShapeDtypeStruct((1, 1), jnp.float32),
        ),
        in_specs=[
            pl.BlockSpec(memory_space=pltpu.VMEM),
            pl.BlockSpec(memory_space=pltpu.VMEM),
            pl.BlockSpec(memory_space=pltpu.VMEM),
        ],
        out_specs=(
            pl.BlockSpec(memory_space=pltpu.VMEM),
            pl.BlockSpec(memory_space=pltpu.VMEM),
            pl.BlockSpec(memory_space=pltpu.VMEM),
            pl.BlockSpec(memory_space=pltpu.SMEM),
        ),
    )(h, w_fc, w_attn)


def _tc_s3_body(e_ref, weatt_ref, wattn_ref, s3_ref, m_ref):
    w3 = wattn_ref[:, 256:272]
    v3 = lax.dot_general(w3, weatt_ref[...], (((1,), (0,)), ((), ())),
                         preferred_element_type=jnp.float32)
    s3 = lax.dot_general(e_ref[...], v3, (((1,), (1,)), ((), ())),
                         preferred_element_type=jnp.float32)
    s3_ref[...] = s3
    bm = jnp.max(s3)

    @pl.when(pl.program_id(0) == 0)
    def _():
        m_ref[0, 0] = bm

    @pl.when(pl.program_id(0) > 0)
    def _():
        m_ref[0, 0] = jnp.maximum(m_ref[0, 0], bm)


def _tc_s3(e, w_eatt, w_attn):
    ne = e.shape[0]
    blk = 20000
    grid = ne // blk
    return pl.pallas_call(
        _tc_s3_body,
        grid=(grid,),
        out_shape=(
            jax.ShapeDtypeStruct((ne, 1), jnp.float32),
            jax.ShapeDtypeStruct((1, 1), jnp.float32),
        ),
        in_specs=[
            pl.BlockSpec((blk, 16), lambda i: (i, 0)),
            pl.BlockSpec((16, 16), lambda i: (0, 0)),
            pl.BlockSpec((1, 272), lambda i: (0, 0)),
        ],
        out_specs=(
            pl.BlockSpec((blk, 1), lambda i: (i, 0)),
            pl.BlockSpec(memory_space=pltpu.SMEM, index_map=lambda i: (0, 0),
                         block_shape=(1, 1)),
        ),
    )(e, w_eatt, w_attn)


def _sc_body(n_nodes, n_edges,
             zsp_hbm, ef_hbm, src_hbm, dst_hbm, s1_hbm, s2_hbm, s3_hbm, b_hbm,
             accz_out, accge_out, accd_out,
             s1_v, s2_v, src_v, dst_v, s3_v, b_v,
             zrows, eflat, esT, colidx, pbuf, d16buf, srcidx, dstidx, zflat,
             accz_sh, accge_sh, accd_sh, gsem, ssem):
    cid = lax.axis_index("c")
    sid = lax.axis_index("s")
    ew = n_edges // _NS
    base = sid * ew
    nchunks = ew // _C
    quarter = n_nodes // 4
    qacc = _QACC
    stripe = qacc // _NS
    rs = sid * stripe

    pltpu.sync_copy(s1_hbm, s1_v)
    pltpu.sync_copy(s2_hbm, s2_v)
    pltpu.sync_copy(src_hbm.at[pl.ds(base, ew)], src_v)
    pltpu.sync_copy(dst_hbm.at[pl.ds(base, ew)], dst_v)
    pltpu.sync_copy(s3_hbm.at[pl.ds(base, ew)], s3_v)
    pltpu.sync_copy(b_hbm, b_v)

    zero = jnp.zeros((_L,), jnp.float32)
    bvec = b_v[...]
    iota16 = lax.iota(jnp.int32, _L) * 16

    for q in range(2):
        qi = cid * 2 + q
        nlo = qi * quarter

        def zrow(i, c):
            for j in range(8):
                zrows[i, pl.ds(j * _L, _L)] = zero
            return c

        lax.fori_loop(0, _C, zrow, 0)

        def zfl(i, c):
            zflat[pl.ds(i * _L, _L)] = zero
            return c

        lax.fori_loop(0, (_C * 16) // _L, zfl, 0)

        for j in range(stripe // _C):
            o = rs + j * _C
            pltpu.sync_copy(zrows, accz_sh.at[pl.ds(o, _C)])
            pltpu.sync_copy(zflat, accge_sh.at[pl.ds(o * 16, _C * 16)])
            pltpu.sync_copy(zflat.at[pl.ds(0, _C)], accd_sh.at[pl.ds(o, _C)])

        plsc.subcore_barrier()

        def chunk(k, carry):
            off = k * _C
            edesc = pltpu.async_copy(
                ef_hbm.at[pl.ds((base + off) * 16, _C * 16)], eflat, gsem)
            for v in range(_C // _L):
                sv = src_v[pl.ds(off + v * _L, _L)]
                dv = dst_v[pl.ds(off + v * _L, _L)]
                srcidx[pl.ds(v * _L, _L)] = sv
                dl = dv - nlo
                inq = (dl >= 0) & (dl < quarter)
                dummy = quarter + ((k + v * 8) % (qacc - quarter - 4))
                dl = jnp.where(inq, dl, dummy)
                dstidx[pl.ds(v * _L, _L)] = dl
                d16buf[pl.ds(v * _L, _L)] = dl * 16
                t = (plsc.load_gather(s1_v, [sv])
                     + plsc.load_gather(s2_v, [dv])
                     + s3_v[pl.ds(off + v * _L, _L)])
                t = jnp.where(t >= 0.0, t, 0.01 * t)
                p = jnp.exp(t - bvec)
                p = jnp.where(inq, p, 0.0)
                pbuf[pl.ds(v * _L, _L)] = p
            pltpu.async_copy(zsp_hbm.at[srcidx], zrows, gsem).wait()

            def edge(i, c):
                pi = plsc.load_gather(pbuf, [jnp.full((_L,), i, jnp.int32)])
                for j in range(8):
                    zrows[i, pl.ds(j * _L, _L)] = (
                        zrows[i, pl.ds(j * _L, _L)] * pi)
                return c

            lax.fori_loop(0, _C, edge, 0)
            edesc.wait()
            for v in range(_C // _L):
                pv = pbuf[pl.ds(v * _L, _L)]
                d16 = d16buf[pl.ds(v * _L, _L)]
                for c in range(16):
                    g = plsc.load_gather(eflat, [iota16 + (v * 256 + c)])
                    esT[c, pl.ds(v * _L, _L)] = g * pv
                    colidx[c, pl.ds(v * _L, _L)] = d16 + c
            descs = [
                pltpu.async_copy(zrows, accz_sh.at[dstidx], ssem, add=True),
                pltpu.async_copy(pbuf, accd_sh.at[dstidx], ssem, add=True)]
            for c in range(16):
                descs.append(pltpu.async_copy(
                    esT.at[c], accge_sh.at[colidx.at[c]], ssem, add=True))
            for d in descs:
                d.wait()
            return carry

        lax.fori_loop(0, nchunks, chunk, 0)

        plsc.subcore_barrier()

        for j in range(stripe // _C):
            o = rs + j * _C
            pltpu.sync_copy(accz_sh.at[pl.ds(o, _C)], zrows)
            pltpu.sync_copy(zrows, accz_out.at[qi, pl.ds(o, _C)])
            pltpu.sync_copy(accge_sh.at[pl.ds(o * 16, _C * 16)], zflat)
            pltpu.sync_copy(
                zflat, accge_out.at[pl.ds((qi * qacc + o) * 16, _C * 16)])
            pltpu.sync_copy(accd_sh.at[pl.ds(o, _C)], pbuf)
            pltpu.sync_copy(pbuf, accd_out.at[pl.ds(qi * qacc + o, _C)])


def _sc_edge_pass(z, e, src, dst, s1, s2, s3, bvec):
    n = z.shape[0]
    ne = e.shape[0]
    ew = ne // _NS
    mesh = plsc.VectorSubcoreMesh(core_axis_name="c", subcore_axis_name="s",
                                  num_cores=_NC, num_subcores=_NS)
    k = functools.partial(
        pl.kernel,
        out_type=(
            jax.ShapeDtypeStruct((4, _QACC, 128), jnp.float32),
            jax.ShapeDtypeStruct((4 * _QACC * 16,), jnp.float32),
            jax.ShapeDtypeStruct((4 * _QACC,), jnp.float32),
        ),
        mesh=mesh,
        compiler_params=pltpu.CompilerParams(needs_layout_passes=False),
        scratch_types=[
            pltpu.VMEM((n,), jnp.float32),
            pltpu.VMEM((n,), jnp.float32),
            pltpu.VMEM((ew,), jnp.int32),
            pltpu.VMEM((ew,), jnp.int32),
            pltpu.VMEM((ew,), jnp.float32),
            pltpu.VMEM((_L,), jnp.float32),
            pltpu.VMEM((_C, 128), jnp.float32),
            pltpu.VMEM((_C * 16,), jnp.float32),
            pltpu.VMEM((16, _C), jnp.float32),
            pltpu.VMEM((16, _C), jnp.int32),
            pltpu.VMEM((_C,), jnp.float32),
            pltpu.VMEM((_C,), jnp.int32),
            pltpu.VMEM((_C,), jnp.int32),
            pltpu.VMEM((_C,), jnp.int32),
            pltpu.VMEM((_C * 16,), jnp.float32),
            pltpu.VMEM_SHARED((_QACC, 128), jnp.float32),
            pltpu.VMEM_SHARED((_QACC * 16,), jnp.float32),
            pltpu.VMEM_SHARED((_QACC,), jnp.float32),
            pltpu.SemaphoreType.DMA,
            pltpu.SemaphoreType.DMA,
        ],
    )(functools.partial(_sc_body, n, ne))
    return k(z, e.reshape(-1), src, dst, s1, s2, s3, bvec)


def _tc_fin_body(accz_ref, accge_ref, accd_ref, wez_ref, weatt_ref, out_ref):
    az = accz_ref[...]
    ge = accge_ref[...]
    den = accd_ref[...]
    m = lax.dot_general(wez_ref[...], weatt_ref[...], (((1,), (0,)), ((), ())),
                        preferred_element_type=jnp.float32)
    h2 = lax.dot_general(ge, m, (((1,), (1,)), ((), ())),
                         preferred_element_type=jnp.float32)
    out_ref[...] = (az + h2) / jnp.maximum(den, 1e-9)


def _tc_fin(accz, accge, accd, w_ez, w_eatt):
    n = accz.shape[0]
    return pl.pallas_call(
        _tc_fin_body,
        out_shape=jax.ShapeDtypeStruct((n, 128), jnp.float32),
        in_specs=[pl.BlockSpec(memory_space=pltpu.VMEM)] * 5,
        out_specs=pl.BlockSpec(memory_space=pltpu.VMEM),
    )(accz, accge, accd, w_ez, w_eatt)


def kernel(h, e, edge_index, W_fc, W_attn, W_eatt, W_ez):
    n = h.shape[0]
    src = edge_index[0]
    dst = edge_index[1]

    z, s1, s2, m12 = _tc_pre(h, W_fc, W_attn)
    s3, m3 = _tc_s3(e, W_eatt, W_attn)

    t = m12[0, 0] + m3[0, 0]
    b = jnp.where(t >= 0.0, t, 0.01 * t)
    bvec = jnp.full((_L,), b, jnp.float32)

    accz, accge, accd = _sc_edge_pass(
        z, e, src, dst,
        s1.reshape(n), s2.reshape(n), s3.reshape(e.shape[0]), bvec)
    nq = n // 4
    az = accz[:, :nq, :].reshape(n, 128)
    ge = accge.reshape(4, _QACC, 16)[:, :nq, :].reshape(n, 16)
    den = accd.reshape(4, _QACC)[:, :nq].reshape(n, 1)
    return _tc_fin(az, ge, den, W_ez, W_eatt)
